# baseline (speedup 1.0000x reference)
import functools

import jax
import jax.numpy as jnp
from jax import lax
from jax.experimental import pallas as pl
from jax.experimental.pallas import tpu as pltpu
from jax.experimental.pallas import tpu_sc as plsc

VOCAB = 2
WIDTH = 1024
N_ROWS = 4 * 8192

NUM_CORES = 2
NUM_SUBCORES = 16
NUM_WORKERS = NUM_CORES * NUM_SUBCORES
ROWS_PER_WORKER = N_ROWS // NUM_WORKERS
CHUNK = 64
NUM_CHUNKS = ROWS_PER_WORKER // CHUNK


@functools.partial(
    pl.kernel,
    out_type=jax.ShapeDtypeStruct((N_ROWS, WIDTH), jnp.float32),
    mesh=plsc.VectorSubcoreMesh(
        core_axis_name="c", subcore_axis_name="s",
        num_cores=NUM_CORES, num_subcores=NUM_SUBCORES,
    ),
    scratch_types=[
        pltpu.VMEM((ROWS_PER_WORKER,), jnp.int32),
        pltpu.VMEM((CHUNK, WIDTH), jnp.float32),
        pltpu.SemaphoreType.DMA,
    ],
)
def _embed_sc(ids_hbm, table_hbm, out_hbm, idx_v, rows_v, sem):
    wid = lax.axis_index("s") * NUM_CORES + lax.axis_index("c")
    base = wid * ROWS_PER_WORKER
    pltpu.sync_copy(ids_hbm.at[pl.ds(base, ROWS_PER_WORKER)], idx_v)

    def chunk_body(i, carry):
        off = pl.multiple_of(i * CHUNK, 8)
        pltpu.async_copy(
            table_hbm.at[idx_v.at[pl.ds(off, CHUNK)]], rows_v, sem
        ).wait()
        pltpu.sync_copy(rows_v, out_hbm.at[pl.ds(base + off, CHUNK)])
        return carry

    lax.fori_loop(0, NUM_CHUNKS, chunk_body, 0)


def kernel(input, kernel):
    ids = jnp.reshape(input, (N_ROWS,)).astype(jnp.int32)
    out = _embed_sc(ids, kernel)
    return jnp.reshape(out, (4, 8192, WIDTH))


# speedup vs baseline: 6.6906x; 6.6906x over previous
import functools

import jax
import jax.numpy as jnp
from jax import lax
from jax.experimental import pallas as pl
from jax.experimental.pallas import tpu as pltpu
from jax.experimental.pallas import tpu_sc as plsc

VOCAB = 2
WIDTH = 1024
N_ROWS = 4 * 8192

NUM_CORES = 2
NUM_SUBCORES = 16
NUM_WORKERS = NUM_CORES * NUM_SUBCORES
ROWS_PER_WORKER = N_ROWS // NUM_WORKERS
NSEM = 8


@functools.partial(
    pl.kernel,
    out_type=jax.ShapeDtypeStruct((N_ROWS, WIDTH), jnp.float32),
    mesh=plsc.VectorSubcoreMesh(
        core_axis_name="c", subcore_axis_name="s",
        num_cores=NUM_CORES, num_subcores=NUM_SUBCORES,
    ),
    scratch_types=[
        pltpu.VMEM((ROWS_PER_WORKER,), jnp.int32),
        pltpu.VMEM((VOCAB, WIDTH), jnp.float32),
        [pltpu.SemaphoreType.DMA] * NSEM,
    ],
)
def _embed_sc(ids_hbm, table_hbm, out_hbm, idx_v, table_v, sems):
    wid = lax.axis_index("s") * NUM_CORES + lax.axis_index("c")
    base = wid * ROWS_PER_WORKER
    pltpu.sync_copy(ids_hbm.at[pl.ds(base, ROWS_PER_WORKER)], idx_v)
    pltpu.sync_copy(table_hbm, table_v)

    def body(s, carry):
        off = pl.multiple_of(s * 16, 16)
        ids16 = idx_v[pl.ds(off, 16)]
        for j in range(16):
            row_id = ids16[j]
            pltpu.async_copy(
                table_v.at[pl.ds(row_id, 1)],
                out_hbm.at[pl.ds(base + off + j, 1)],
                sems[j % NSEM],
            ).wait()
        return carry

    lax.fori_loop(0, ROWS_PER_WORKER // 16, body, 0)


def kernel(input, kernel):
    ids = jnp.reshape(input, (N_ROWS,)).astype(jnp.int32)
    out = _embed_sc(ids, kernel)
    return jnp.reshape(out, (4, 8192, WIDTH))


# speedup vs baseline: 13.1748x; 1.9692x over previous
import functools

import jax
import jax.numpy as jnp
from jax import lax
from jax.experimental import pallas as pl
from jax.experimental.pallas import tpu as pltpu
from jax.experimental.pallas import tpu_sc as plsc

VOCAB = 2
WIDTH = 1024
N_ROWS = 4 * 8192

NUM_CORES = 2
NUM_SUBCORES = 16
NUM_WORKERS = NUM_CORES * NUM_SUBCORES
ROWS_PER_WORKER = N_ROWS // NUM_WORKERS
NSEM = 16


@functools.partial(
    pl.kernel,
    out_type=jax.ShapeDtypeStruct((N_ROWS, WIDTH), jnp.float32),
    mesh=plsc.VectorSubcoreMesh(
        core_axis_name="c", subcore_axis_name="s",
        num_cores=NUM_CORES, num_subcores=NUM_SUBCORES,
    ),
    scratch_types=[
        pltpu.VMEM((ROWS_PER_WORKER,), jnp.int32),
        pltpu.VMEM((VOCAB, WIDTH), jnp.float32),
        [pltpu.SemaphoreType.DMA] * NSEM,
    ],
)
def _embed_sc(ids_hbm, table_hbm, out_hbm, idx_v, table_v, sems):
    wid = lax.axis_index("s") * NUM_CORES + lax.axis_index("c")
    base = wid * ROWS_PER_WORKER
    pltpu.sync_copy(ids_hbm.at[pl.ds(base, ROWS_PER_WORKER)], idx_v)
    pltpu.sync_copy(table_hbm, table_v)

    def issue_slice(s, *, drain_first):
        off = pl.multiple_of(s * 16, 16)
        ids16 = idx_v[pl.ds(off, 16)]
        for j in range(16):
            if drain_first:
                pltpu.make_async_copy(
                    table_v.at[pl.ds(0, 1)],
                    out_hbm.at[pl.ds(base, 1)],
                    sems[j],
                ).wait()
            row_id = ids16[j]
            pltpu.async_copy(
                table_v.at[pl.ds(row_id, 1)],
                out_hbm.at[pl.ds(base + off + j, 1)],
                sems[j],
            )

    issue_slice(0, drain_first=False)

    def body(s, carry):
        issue_slice(s, drain_first=True)
        return carry

    lax.fori_loop(1, ROWS_PER_WORKER // 16, body, 0)

    for j in range(16):
        pltpu.make_async_copy(
            table_v.at[pl.ds(0, 1)],
            out_hbm.at[pl.ds(base, 1)],
            sems[j],
        ).wait()


def kernel(input, kernel):
    ids = jnp.reshape(input, (N_ROWS,)).astype(jnp.int32)
    out = _embed_sc(ids, kernel)
    return jnp.reshape(out, (4, 8192, WIDTH))
